# baseline (speedup 1.0000x reference)
import jax
import jax.numpy as jnp
from jax.experimental import pallas as pl
from jax.experimental.pallas import tpu as pltpu

NUM_EXPERTS = 8
D_MODEL = 768
T_TILE = 1024


def _moe_body(logits_ref, x_ref, w_ref, b_ref, out_ref, wb_ref):
    @pl.when(pl.program_id(0) == 0)
    def _cast_w():
        wb_ref[...] = w_ref[...].astype(jnp.bfloat16)

    logits = logits_ref[...]
    x = x_ref[...]
    m1 = jnp.max(logits, axis=-1, keepdims=True)
    i1 = jnp.argmax(logits, axis=-1)[:, None]
    masked = jnp.where(jax.lax.broadcasted_iota(jnp.int32, logits.shape, 1) == i1,
                       jnp.full_like(logits, -jnp.inf), logits)
    m2 = jnp.max(masked, axis=-1, keepdims=True)
    i2 = jnp.argmax(masked, axis=-1)[:, None]
    e2 = jnp.exp(m2 - m1)
    w1 = 1.0 / (1.0 + e2)
    w2 = e2 / (1.0 + e2)
    acc = jnp.zeros_like(x)
    xb = x.astype(jnp.bfloat16)
    for e in range(NUM_EXPERTS):
        ce = jnp.where(i1 == e, w1, jnp.where(i2 == e, w2, 0.0))
        y = jax.lax.dot_general(
            xb, wb_ref[e], (((1,), (0,)), ((), ())),
            preferred_element_type=jnp.float32,
        ) + b_ref[e][None, :]
        acc = acc + ce * y
    out_ref[...] = acc


def kernel(hidden, gate_logits, W_experts, b_experts):
    T, D = hidden.shape
    return pl.pallas_call(
        _moe_body,
        grid=(T // T_TILE,),
        in_specs=[
            pl.BlockSpec((T_TILE, NUM_EXPERTS), lambda i: (i, 0)),
            pl.BlockSpec((T_TILE, D), lambda i: (i, 0)),
            pl.BlockSpec((NUM_EXPERTS, D, D), lambda i: (0, 0, 0)),
            pl.BlockSpec((NUM_EXPERTS, D), lambda i: (0, 0)),
        ],
        out_specs=pl.BlockSpec((T_TILE, D), lambda i: (i, 0)),
        out_shape=jax.ShapeDtypeStruct((T, D), jnp.float32),
        scratch_shapes=[pltpu.VMEM((NUM_EXPERTS, D, D), jnp.bfloat16)],
    )(gate_logits, hidden, W_experts, b_experts)


# speedup vs baseline: 1.0372x; 1.0372x over previous
import jax
import jax.numpy as jnp
from jax.experimental import pallas as pl
from jax.experimental.pallas import tpu as pltpu

NUM_EXPERTS = 8
D_MODEL = 768
T_TILE = 512


def _moe_body(logits_ref, x_ref, w_ref, b_ref, out_ref, wb_ref):
    @pl.when(pl.program_id(0) == 0)
    def _cast_w():
        wb_ref[...] = w_ref[...].astype(jnp.bfloat16)

    logits = logits_ref[...]
    x = x_ref[...]
    m1 = jnp.max(logits, axis=-1, keepdims=True)
    i1 = jnp.argmax(logits, axis=-1)[:, None]
    masked = jnp.where(jax.lax.broadcasted_iota(jnp.int32, logits.shape, 1) == i1,
                       jnp.full_like(logits, -jnp.inf), logits)
    m2 = jnp.max(masked, axis=-1, keepdims=True)
    i2 = jnp.argmax(masked, axis=-1)[:, None]
    e2 = jnp.exp(m2 - m1)
    w1 = 1.0 / (1.0 + e2)
    w2 = e2 / (1.0 + e2)
    acc = jnp.zeros_like(x)
    xb = x.astype(jnp.bfloat16)
    for e in range(NUM_EXPERTS):
        ce = jnp.where(i1 == e, w1, jnp.where(i2 == e, w2, 0.0))
        y = jax.lax.dot_general(
            xb, wb_ref[e], (((1,), (0,)), ((), ())),
            preferred_element_type=jnp.float32,
        ) + b_ref[e][None, :]
        acc = acc + ce * y
    out_ref[...] = acc


def kernel(hidden, gate_logits, W_experts, b_experts):
    T, D = hidden.shape
    return pl.pallas_call(
        _moe_body,
        grid=(T // T_TILE,),
        in_specs=[
            pl.BlockSpec((T_TILE, NUM_EXPERTS), lambda i: (i, 0)),
            pl.BlockSpec((T_TILE, D), lambda i: (i, 0)),
            pl.BlockSpec((NUM_EXPERTS, D, D), lambda i: (0, 0, 0)),
            pl.BlockSpec((NUM_EXPERTS, D), lambda i: (0, 0)),
        ],
        out_specs=pl.BlockSpec((T_TILE, D), lambda i: (i, 0)),
        out_shape=jax.ShapeDtypeStruct((T, D), jnp.float32),
        scratch_shapes=[pltpu.VMEM((NUM_EXPERTS, D, D), jnp.bfloat16)],
    )(gate_logits, hidden, W_experts, b_experts)
